# baseline (speedup 1.0000x reference)
import functools

import jax
import jax.numpy as jnp
from jax import lax
from jax.experimental import pallas as pl
from jax.experimental.pallas import tpu as pltpu
from jax.experimental.pallas import tpu_sc as plsc

NMS_THR = 0.5
EPS = 1e-06
PULL_W = 1.0
PUSH_W = 1.0

L = 16
NTILES = 16
NEG_INF = float("-inf")
LN2 = 0.6931471805599453
SQRT2 = 1.4142135623730951


def _logv(x):
    bits = lax.bitcast_convert_type(x, jnp.int32)
    e = ((bits >> 23) & 0xFF) - 127
    m = lax.bitcast_convert_type((bits & 0x007FFFFF) | 0x3F800000, jnp.float32)
    big = m > SQRT2
    m = jnp.where(big, m * 0.5, m)
    ef = (e + jnp.where(big, 1, 0)).astype(jnp.float32)
    s = (m - 1.0) / (m + 1.0)
    z = s * s
    p = s * (2.0 + z * (0.6666666666666666 + z * (0.4 + z * 0.2857142857142857)))
    return ef * LN2 + p


def _sld(ref, i, iota):
    return jnp.max(plsc.load_gather(ref, [iota * 0 + i]))


def _sdiv(a, b, zero16):
    return jnp.max((zero16 + a) / (zero16 + b))


def _pack5(iota, a, b, c, d, e):
    v = jnp.where(iota == 0, a, 0.0)
    v = jnp.where(iota == 1, b, v)
    v = jnp.where(iota == 2, c, v)
    v = jnp.where(iota == 3, d, v)
    v = jnp.where(iota == 4, e, v)
    return v


def _build(P, PP, G, C, GIP):
    SL = PP // NTILES
    NCH = SL // L
    mesh = plsc.VectorSubcoreMesh(core_axis_name="c", subcore_axis_name="s")

    @functools.partial(
        pl.kernel,
        out_type=jax.ShapeDtypeStruct((2, L), jnp.float32),
        mesh=mesh,
        scratch_types=[
            pltpu.VMEM((PP,), jnp.float32),
            pltpu.VMEM((PP,), jnp.float32),
            pltpu.VMEM((PP,), jnp.float32),
            pltpu.VMEM((PP,), jnp.float32),
            pltpu.VMEM((PP,), jnp.float32),
            pltpu.VMEM((PP,), jnp.float32),
            pltpu.VMEM((PP,), jnp.int32),
            pltpu.VMEM((SL,), jnp.int32),
            pltpu.VMEM((SL,), jnp.float32),
            pltpu.VMEM((SL, C), jnp.float32),
            pltpu.VMEM((256,), jnp.float32),
            pltpu.VMEM((64,), jnp.int32),
            pltpu.VMEM((GIP,), jnp.float32),
            pltpu.VMEM((64,), jnp.int32),
            pltpu.VMEM((L,), jnp.float32),
            pltpu.VMEM((NTILES * L,), jnp.float32),
            pltpu.VMEM_SHARED((PP,), jnp.float32),
            pltpu.VMEM_SHARED((NTILES * L,), jnp.float32),
        ],
        compiler_params=pltpu.CompilerParams(needs_layout_passes=False),
    )
    def nms_kernel(x1h, y1h, x2h, y2h, clsh, gtih, gtfh, gtlh, outh,
                   fx1, fy1, fx2, fy2, fscore, farea, fgti, sact, sscore, scls,
                   sgtf, sgtl, sgtiou, maxrec, prow, pall, shscore, shpart):
        cimg = lax.axis_index("c")
        sid = lax.axis_index("s")
        base = sid * SL
        iota = lax.iota(jnp.int32, L)
        zero16 = jnp.zeros((L,), jnp.float32)
        ninf16 = zero16 + NEG_INF

        pltpu.sync_copy(x1h.at[cimg], fx1)
        pltpu.sync_copy(y1h.at[cimg], fy1)
        pltpu.sync_copy(x2h.at[cimg], fx2)
        pltpu.sync_copy(y2h.at[cimg], fy2)
        pltpu.sync_copy(gtih.at[cimg], fgti)
        pltpu.sync_copy(clsh.at[cimg, pl.ds(base, SL)], scls)
        pltpu.sync_copy(gtfh.at[cimg], sgtf)
        pltpu.sync_copy(gtlh.at[cimg], sgtl)

        for k in range(4):
            maxrec[pl.ds(k * L, L)] = jnp.zeros((L,), jnp.int32) - 1

        def gt_body(t, carry):
            idx = t * L + iota
            i_ = lax.div(idx, G)
            j_ = idx - i_ * G
            i4 = i_ * 4
            j4 = j_ * 4
            ax1 = plsc.load_gather(sgtf, [i4])
            ay1 = plsc.load_gather(sgtf, [i4 + 1])
            ax2 = plsc.load_gather(sgtf, [i4 + 2])
            ay2 = plsc.load_gather(sgtf, [i4 + 3])
            bx1 = plsc.load_gather(sgtf, [j4])
            by1 = plsc.load_gather(sgtf, [j4 + 1])
            bx2 = plsc.load_gather(sgtf, [j4 + 2])
            by2 = plsc.load_gather(sgtf, [j4 + 3])
            ltx = jnp.maximum(ax1, bx1)
            lty = jnp.maximum(ay1, by1)
            rbx = jnp.minimum(ax2, bx2)
            rby = jnp.minimum(ay2, by2)
            whx = jnp.maximum(rbx - ltx + 1.0, 0.0)
            why = jnp.maximum(rby - lty + 1.0, 0.0)
            ovl = whx * why
            a1 = (ax2 - ax1 + 1.0) * (ay2 - ay1 + 1.0)
            a2 = (bx2 - bx1 + 1.0) * (by2 - by1 + 1.0)
            sgtiou[pl.ds(t * L, L)] = ovl / (a1 + a2 - ovl)
            return carry
        lax.fori_loop(0, GIP // L, gt_body, jnp.int32(0))

        def sc_body(t, carry):
            runv, runi = carry
            lidx = t * L + iota
            gidx = base + lidx
            gtic = fgti[pl.ds(base + t * L, L)]
            labels = plsc.load_gather(sgtl, [gtic])
            srow = plsc.load_gather(scls, [lidx, labels])
            sscore[pl.ds(t * L, L)] = srow
            xc = fx1[pl.ds(base + t * L, L)]
            yc = fy1[pl.ds(base + t * L, L)]
            x2c = fx2[pl.ds(base + t * L, L)]
            y2c = fy2[pl.ds(base + t * L, L)]
            farea[pl.ds(base + t * L, L)] = (x2c - xc + 1.0) * (y2c - yc + 1.0)
            actb = gidx < P
            sact[pl.ds(t * L, L)] = jnp.where(actb, 1, 0)
            cval = jnp.where(actb, srow, NEG_INF)
            take = cval >= runv
            runv = jnp.maximum(runv, cval)
            runi = jnp.where(take, gidx.astype(jnp.float32), runi)
            return runv, runi
        runv, runi = lax.fori_loop(0, NCH, sc_body, (ninf16, zero16 - 1.0))

        mval = jnp.max(runv)
        midx = jnp.max(jnp.where(runv == mval, runi, -1.0))
        prow[...] = _pack5(iota, mval, midx, 0.0, 0.0, 0.0)
        pltpu.sync_copy(sscore, shscore.at[pl.ds(base, SL)])
        pltpu.sync_copy(prow, shpart.at[pl.ds(sid * L, L)])
        plsc.subcore_barrier()
        pltpu.sync_copy(shscore, fscore)

        def phase2(tpush, tpull, pcntf, plcnt, ac):
            pltpu.sync_copy(shpart, pall)
            plsc.subcore_barrier()
            mv = plsc.load_gather(pall, [iota * L])
            mi = plsc.load_gather(pall, [iota * L + 1])
            ps = plsc.load_gather(pall, [iota * L + 2])
            cn = plsc.load_gather(pall, [iota * L + 3])
            ov = plsc.load_gather(pall, [iota * L + 4])
            gmval = jnp.max(mv)
            i_f = jnp.max(jnp.where(mv == gmval, mi, -1.0))
            push_sum = jnp.sum(ps)
            cntf = jnp.sum(cn)
            ovf = jnp.sum(ov)
            tpush = tpush + jnp.where(cntf > 0,
                                      _sdiv(push_sum, jnp.maximum(cntf, 1.0),
                                            zero16), 0.0)
            pcntf = pcntf + cntf
            ac = ac - ovf.astype(jnp.int32)
            flag = ac > 0
            inew = jnp.maximum(i_f, 0.0).astype(jnp.int32)
            g = _sld(fgti, inew, iota)
            rec = _sld(maxrec, g, iota)
            has = rec >= 0
            xi1 = _sld(fx1, inew, iota)
            yi1 = _sld(fy1, inew, iota)
            xi2 = _sld(fx2, inew, iota)
            yi2 = _sld(fy2, inew, iota)
            area_i = (xi2 - xi1 + 1.0) * (yi2 - yi1 + 1.0)
            recc = jnp.where(has, rec, 0)
            xr1 = _sld(fx1, recc, iota)
            yr1 = _sld(fy1, recc, iota)
            xr2 = _sld(fx2, recc, iota)
            yr2 = _sld(fy2, recc, iota)
            ltx = jnp.maximum(xr1, xi1)
            lty = jnp.maximum(yr1, yi1)
            rbx = jnp.minimum(xr2, xi2)
            rby = jnp.minimum(yr2, yi2)
            whx = jnp.maximum(rbx - ltx + 1.0, 0.0)
            why = jnp.maximum(rby - lty + 1.0, 0.0)
            ovlp = whx * why
            area_r = (xr2 - xr1 + 1.0) * (yr2 - yr1 + 1.0)
            pr = _sdiv(ovlp, area_r + area_i - ovlp, zero16)
            msiou = jnp.maximum(pr, EPS)
            lgs = jnp.max(_logv(zero16 + ((1.0 - NMS_THR) + msiou)))
            pullv = -lgs * _sld(fscore, inew, iota)
            rem = (ac - 1) > 0
            tpull = tpull + jnp.where(flag & has & rem, pullv, 0.0)
            plcnt = plcnt + jnp.where(flag & has, 1, 0)
            newrec = jnp.where(flag & (~has), inew, rec)
            plsc.store_scatter(maxrec, [iota * 0 + g], iota * 0 + newrec,
                               mask=iota == 0)
            return (flag.astype(jnp.int32), g, xi1, yi1, xi2, yi2, area_i,
                    tpush, tpull, pcntf, plcnt, ac)

        f32z = jnp.float32(0.0)
        carry0 = phase2(f32z, f32z, f32z, jnp.int32(0), jnp.int32(P))

        def loop_body(carry):
            (cf, g, xi1, yi1, xi2, yi2, area_i,
             tpush, tpull, pcntf, plcnt, ac, it) = carry
            g50 = g * G

            def ch_body(t, ch):
                runv, runi, pacc, cacc, oacc = ch
                o = base + t * L
                gidx = o + iota
                xc = fx1[pl.ds(o, L)]
                yc = fy1[pl.ds(o, L)]
                x2c = fx2[pl.ds(o, L)]
                y2c = fy2[pl.ds(o, L)]
                sc = fscore[pl.ds(o, L)]
                gtic = fgti[pl.ds(o, L)]
                act = sact[pl.ds(t * L, L)]
                ltx = jnp.maximum(xc, xi1)
                lty = jnp.maximum(yc, yi1)
                rbx = jnp.minimum(x2c, xi2)
                rby = jnp.minimum(y2c, yi2)
                whx = jnp.maximum(rbx - ltx + 1.0, 0.0)
                why = jnp.maximum(rby - lty + 1.0, 0.0)
                ovl = whx * why
                a2 = farea[pl.ds(o, L)]
                cur = ovl / (area_i + a2 - ovl)
                actb = act > 0
                ovb = actb & (cur > NMS_THR)
                gmat = plsc.load_gather(sgtiou, [g50 + gtic])
                pm2 = ovb & (gtic != g) & (cur > gmat)
                lg = _logv(1.0 - cur)
                plv = -lg * sc
                pacc = pacc + jnp.where(pm2, plv, 0.0)
                cacc = cacc + jnp.where(pm2, 1.0, 0.0)
                oacc = oacc + jnp.where(ovb, 1.0, 0.0)
                nact = jnp.where(ovb, 0, act)
                sact[pl.ds(t * L, L)] = nact
                cval = jnp.where(nact > 0, sc, NEG_INF)
                take = cval >= runv
                runv = jnp.maximum(runv, cval)
                runi = jnp.where(take, gidx.astype(jnp.float32), runi)
                return runv, runi, pacc, cacc, oacc

            def ch2_body(t2, ch):
                ch = ch_body(t2 * 2, ch)
                return ch_body(t2 * 2 + 1, ch)

            runv, runi, pacc, cacc, oacc = lax.fori_loop(
                0, NCH // 2, ch2_body,
                (ninf16, zero16 - 1.0, zero16, zero16, zero16))
            mval = jnp.max(runv)
            midx = jnp.max(jnp.where(runv == mval, runi, -1.0))
            prow[...] = _pack5(iota, mval, midx, jnp.sum(pacc),
                               jnp.sum(cacc), jnp.sum(oacc))
            pltpu.sync_copy(prow, shpart.at[pl.ds(sid * L, L)])
            plsc.subcore_barrier()
            return phase2(tpush, tpull, pcntf, plcnt, ac) + (it + 1,)

        fin = lax.while_loop(lambda cr: (cr[0] > 0) & (cr[12] < PP),
                             loop_body, carry0 + (jnp.int32(0),))
        tpush, tpull, pcntf, plcnt = fin[7], fin[8], fin[9], fin[10]

        push_v = (zero16 + tpush) / (zero16 + (pcntf + EPS))
        pull_v = (zero16 + tpull) / (zero16 + (plcnt.astype(jnp.float32) + EPS))

        @pl.when(sid == 0)
        def _():
            prow[...] = jnp.where(iota == 0, push_v,
                                  jnp.where(iota == 1, pull_v, 0.0))
            pltpu.sync_copy(prow, outh.at[cimg])

    return nms_kernel


@jax.jit
def kernel(pos_inds, pos_gt_index, gt_bboxes, bbox_preds, cls_scores, gt_labels):
    IMG, P, C = cls_scores.shape
    G = gt_bboxes.shape[1]
    SL = ((P + NTILES * L - 1) // (NTILES * L)) * L
    PP = SL * NTILES
    GIP = ((G * G + L - 1) // L) * L
    pad = PP - P

    bp = bbox_preds.astype(jnp.float32)
    x1 = jnp.pad(bp[..., 0], ((0, 0), (0, pad)))
    y1 = jnp.pad(bp[..., 1], ((0, 0), (0, pad)))
    x2 = jnp.pad(bp[..., 2], ((0, 0), (0, pad)))
    y2 = jnp.pad(bp[..., 3], ((0, 0), (0, pad)))
    clsp = jnp.pad(cls_scores.astype(jnp.float32), ((0, 0), (0, pad), (0, 0)))
    gti = jnp.pad(pos_gt_index.astype(jnp.int32), ((0, 0), (0, pad)))
    gtf = jnp.pad(gt_bboxes.astype(jnp.float32).reshape(IMG, G * 4),
                  ((0, 0), (0, 256 - G * 4)))
    gtl = jnp.pad(gt_labels.astype(jnp.int32), ((0, 0), (0, 64 - G)))

    out = _build(P, PP, G, C, GIP)(x1, y1, x2, y2, clsp, gti, gtf, gtl)

    push = (0.0 + out[0, 0] + out[1, 0]) / IMG * PUSH_W
    pull = (0.0 + out[0, 1] + out[1, 1]) / IMG * PULL_W
    return jnp.stack([push, pull])


# speedup vs baseline: 1.0367x; 1.0367x over previous
import functools

import jax
import jax.numpy as jnp
from jax import lax
from jax.experimental import pallas as pl
from jax.experimental.pallas import tpu as pltpu
from jax.experimental.pallas import tpu_sc as plsc

NMS_THR = 0.5
EPS = 1e-06
PULL_W = 1.0
PUSH_W = 1.0

L = 16
NTILES = 16
NEG_INF = float("-inf")
LN2 = 0.6931471805599453
SQRT2 = 1.4142135623730951


def _logv(x):
    bits = lax.bitcast_convert_type(x, jnp.int32)
    e = ((bits >> 23) & 0xFF) - 127
    m = lax.bitcast_convert_type((bits & 0x007FFFFF) | 0x3F800000, jnp.float32)
    big = m > SQRT2
    m = jnp.where(big, m * 0.5, m)
    ef = (e + jnp.where(big, 1, 0)).astype(jnp.float32)
    s = (m - 1.0) / (m + 1.0)
    z = s * s
    p = s * (2.0 + z * (0.6666666666666666 + z * (0.4 + z * 0.2857142857142857)))
    return ef * LN2 + p


def _sld(ref, i, iota):
    return jnp.max(plsc.load_gather(ref, [iota * 0 + i]))


def _sdiv(a, b, zero16):
    return jnp.max((zero16 + a) / (zero16 + b))


def _pack5(iota, a, b, c, d, e):
    v = jnp.where(iota == 0, a, 0.0)
    v = jnp.where(iota == 1, b, v)
    v = jnp.where(iota == 2, c, v)
    v = jnp.where(iota == 3, d, v)
    v = jnp.where(iota == 4, e, v)
    return v


def _build(P, PP, G, C, GIP):
    SL = PP // NTILES
    NCH = SL // L
    mesh = plsc.VectorSubcoreMesh(core_axis_name="c", subcore_axis_name="s")

    @functools.partial(
        pl.kernel,
        out_type=jax.ShapeDtypeStruct((2, L), jnp.float32),
        mesh=mesh,
        scratch_types=[
            pltpu.VMEM((PP,), jnp.float32),
            pltpu.VMEM((PP,), jnp.float32),
            pltpu.VMEM((PP,), jnp.float32),
            pltpu.VMEM((PP,), jnp.float32),
            pltpu.VMEM((PP,), jnp.float32),
            pltpu.VMEM((PP,), jnp.float32),
            pltpu.VMEM((PP,), jnp.int32),
            pltpu.VMEM((SL,), jnp.int32),
            pltpu.VMEM((SL,), jnp.float32),
            pltpu.VMEM((SL, C), jnp.float32),
            pltpu.VMEM((256,), jnp.float32),
            pltpu.VMEM((64,), jnp.int32),
            pltpu.VMEM((GIP,), jnp.float32),
            pltpu.VMEM((64,), jnp.int32),
            pltpu.VMEM((L,), jnp.float32),
            pltpu.VMEM((NTILES * L,), jnp.float32),
            pltpu.VMEM_SHARED((PP,), jnp.float32),
            pltpu.VMEM_SHARED((NTILES * L,), jnp.float32),
            pltpu.VMEM_SHARED((NTILES * L,), jnp.float32),
        ],
        compiler_params=pltpu.CompilerParams(needs_layout_passes=False),
    )
    def nms_kernel(x1h, y1h, x2h, y2h, clsh, gtih, gtfh, gtlh, outh,
                   fx1, fy1, fx2, fy2, fscore, farea, fgti, sact, sscore, scls,
                   sgtf, sgtl, sgtiou, maxrec, prow, pall, shscore, shpart,
                   shpart2):
        cimg = lax.axis_index("c")
        sid = lax.axis_index("s")
        base = sid * SL
        iota = lax.iota(jnp.int32, L)
        zero16 = jnp.zeros((L,), jnp.float32)
        ninf16 = zero16 + NEG_INF

        pltpu.sync_copy(x1h.at[cimg], fx1)
        pltpu.sync_copy(y1h.at[cimg], fy1)
        pltpu.sync_copy(x2h.at[cimg], fx2)
        pltpu.sync_copy(y2h.at[cimg], fy2)
        pltpu.sync_copy(gtih.at[cimg], fgti)
        pltpu.sync_copy(clsh.at[cimg, pl.ds(base, SL)], scls)
        pltpu.sync_copy(gtfh.at[cimg], sgtf)
        pltpu.sync_copy(gtlh.at[cimg], sgtl)

        for k in range(4):
            maxrec[pl.ds(k * L, L)] = jnp.zeros((L,), jnp.int32) - 1

        def gt_body(t, carry):
            idx = t * L + iota
            i_ = lax.div(idx, G)
            j_ = idx - i_ * G
            i4 = i_ * 4
            j4 = j_ * 4
            ax1 = plsc.load_gather(sgtf, [i4])
            ay1 = plsc.load_gather(sgtf, [i4 + 1])
            ax2 = plsc.load_gather(sgtf, [i4 + 2])
            ay2 = plsc.load_gather(sgtf, [i4 + 3])
            bx1 = plsc.load_gather(sgtf, [j4])
            by1 = plsc.load_gather(sgtf, [j4 + 1])
            bx2 = plsc.load_gather(sgtf, [j4 + 2])
            by2 = plsc.load_gather(sgtf, [j4 + 3])
            ltx = jnp.maximum(ax1, bx1)
            lty = jnp.maximum(ay1, by1)
            rbx = jnp.minimum(ax2, bx2)
            rby = jnp.minimum(ay2, by2)
            whx = jnp.maximum(rbx - ltx + 1.0, 0.0)
            why = jnp.maximum(rby - lty + 1.0, 0.0)
            ovl = whx * why
            a1 = (ax2 - ax1 + 1.0) * (ay2 - ay1 + 1.0)
            a2 = (bx2 - bx1 + 1.0) * (by2 - by1 + 1.0)
            sgtiou[pl.ds(t * L, L)] = ovl / (a1 + a2 - ovl)
            return carry
        lax.fori_loop(0, GIP // L, gt_body, jnp.int32(0))

        def sc_body(t, carry):
            runv, runi = carry
            lidx = t * L + iota
            gidx = base + lidx
            gtic = fgti[pl.ds(base + t * L, L)]
            labels = plsc.load_gather(sgtl, [gtic])
            srow = plsc.load_gather(scls, [lidx, labels])
            sscore[pl.ds(t * L, L)] = srow
            xc = fx1[pl.ds(base + t * L, L)]
            yc = fy1[pl.ds(base + t * L, L)]
            x2c = fx2[pl.ds(base + t * L, L)]
            y2c = fy2[pl.ds(base + t * L, L)]
            farea[pl.ds(base + t * L, L)] = (x2c - xc + 1.0) * (y2c - yc + 1.0)
            actb = gidx < P
            sact[pl.ds(t * L, L)] = jnp.where(actb, 1, 0)
            cval = jnp.where(actb, srow, NEG_INF)
            take = cval >= runv
            runv = jnp.maximum(runv, cval)
            runi = jnp.where(take, gidx.astype(jnp.float32), runi)
            return runv, runi
        runv, runi = lax.fori_loop(0, NCH, sc_body, (ninf16, zero16 - 1.0))

        mval = jnp.max(runv)
        midx = jnp.max(jnp.where(runv == mval, runi, -1.0))
        prow[...] = _pack5(iota, mval, midx, 0.0, 0.0, 0.0)
        pltpu.sync_copy(sscore, shscore.at[pl.ds(base, SL)])
        pltpu.sync_copy(prow, shpart.at[pl.ds(sid * L, L)])
        plsc.subcore_barrier()
        pltpu.sync_copy(shscore, fscore)

        def phase2(sh, tpush, tpull, pcntf, plcnt, ac):
            pltpu.sync_copy(sh, pall)
            mv = plsc.load_gather(pall, [iota * L])
            mi = plsc.load_gather(pall, [iota * L + 1])
            ps = plsc.load_gather(pall, [iota * L + 2])
            cn = plsc.load_gather(pall, [iota * L + 3])
            ov = plsc.load_gather(pall, [iota * L + 4])
            gmval = jnp.max(mv)
            i_f = jnp.max(jnp.where(mv == gmval, mi, -1.0))
            push_sum = jnp.sum(ps)
            cntf = jnp.sum(cn)
            ovf = jnp.sum(ov)
            tpush = tpush + jnp.where(cntf > 0,
                                      _sdiv(push_sum, jnp.maximum(cntf, 1.0),
                                            zero16), 0.0)
            pcntf = pcntf + cntf
            ac = ac - ovf.astype(jnp.int32)
            flag = ac > 0
            inew = jnp.maximum(i_f, 0.0).astype(jnp.int32)
            g = _sld(fgti, inew, iota)
            rec = _sld(maxrec, g, iota)
            has = rec >= 0
            xi1 = _sld(fx1, inew, iota)
            yi1 = _sld(fy1, inew, iota)
            xi2 = _sld(fx2, inew, iota)
            yi2 = _sld(fy2, inew, iota)
            area_i = (xi2 - xi1 + 1.0) * (yi2 - yi1 + 1.0)
            recc = jnp.where(has, rec, 0)
            xr1 = _sld(fx1, recc, iota)
            yr1 = _sld(fy1, recc, iota)
            xr2 = _sld(fx2, recc, iota)
            yr2 = _sld(fy2, recc, iota)
            ltx = jnp.maximum(xr1, xi1)
            lty = jnp.maximum(yr1, yi1)
            rbx = jnp.minimum(xr2, xi2)
            rby = jnp.minimum(yr2, yi2)
            whx = jnp.maximum(rbx - ltx + 1.0, 0.0)
            why = jnp.maximum(rby - lty + 1.0, 0.0)
            ovlp = whx * why
            area_r = (xr2 - xr1 + 1.0) * (yr2 - yr1 + 1.0)
            pr = _sdiv(ovlp, area_r + area_i - ovlp, zero16)
            msiou = jnp.maximum(pr, EPS)
            lgs = jnp.max(_logv(zero16 + ((1.0 - NMS_THR) + msiou)))
            pullv = -lgs * _sld(fscore, inew, iota)
            rem = (ac - 1) > 0
            tpull = tpull + jnp.where(flag & has & rem, pullv, 0.0)
            plcnt = plcnt + jnp.where(flag & has, 1, 0)
            newrec = jnp.where(flag & (~has), inew, rec)
            plsc.store_scatter(maxrec, [iota * 0 + g], iota * 0 + newrec,
                               mask=iota == 0)
            return (flag.astype(jnp.int32), g, xi1, yi1, xi2, yi2, area_i,
                    tpush, tpull, pcntf, plcnt, ac)

        f32z = jnp.float32(0.0)
        carry0 = phase2(shpart, f32z, f32z, f32z, jnp.int32(0), jnp.int32(P))

        def half(carry, sh):
            (cf, g, xi1, yi1, xi2, yi2, area_i,
             tpush, tpull, pcntf, plcnt, ac, it) = carry
            g50 = g * G

            def ch_body(t, ch):
                runv, runi, pacc, cacc, oacc = ch
                o = base + t * L
                gidx = o + iota
                xc = fx1[pl.ds(o, L)]
                yc = fy1[pl.ds(o, L)]
                x2c = fx2[pl.ds(o, L)]
                y2c = fy2[pl.ds(o, L)]
                sc = fscore[pl.ds(o, L)]
                gtic = fgti[pl.ds(o, L)]
                act = sact[pl.ds(t * L, L)]
                ltx = jnp.maximum(xc, xi1)
                lty = jnp.maximum(yc, yi1)
                rbx = jnp.minimum(x2c, xi2)
                rby = jnp.minimum(y2c, yi2)
                whx = jnp.maximum(rbx - ltx + 1.0, 0.0)
                why = jnp.maximum(rby - lty + 1.0, 0.0)
                ovl = whx * why
                a2 = farea[pl.ds(o, L)]
                cur = ovl / (area_i + a2 - ovl)
                actb = act > 0
                ovb = actb & (cur > NMS_THR)
                gmat = plsc.load_gather(sgtiou, [g50 + gtic])
                pm2 = ovb & (gtic != g) & (cur > gmat)
                lg = _logv(1.0 - cur)
                plv = -lg * sc
                pacc = pacc + jnp.where(pm2, plv, 0.0)
                cacc = cacc + jnp.where(pm2, 1.0, 0.0)
                oacc = oacc + jnp.where(ovb, 1.0, 0.0)
                nact = jnp.where(ovb, 0, act)
                sact[pl.ds(t * L, L)] = nact
                cval = jnp.where(nact > 0, sc, NEG_INF)
                take = cval >= runv
                runv = jnp.maximum(runv, cval)
                runi = jnp.where(take, gidx.astype(jnp.float32), runi)
                return runv, runi, pacc, cacc, oacc

            def ch2_body(t2, ch):
                ch = ch_body(t2 * 2, ch)
                return ch_body(t2 * 2 + 1, ch)

            runv, runi, pacc, cacc, oacc = lax.fori_loop(
                0, NCH // 2, ch2_body,
                (ninf16, zero16 - 1.0, zero16, zero16, zero16))
            mval = jnp.max(runv)
            midx = jnp.max(jnp.where(runv == mval, runi, -1.0))
            prow[...] = _pack5(iota, mval, midx, jnp.sum(pacc),
                               jnp.sum(cacc), jnp.sum(oacc))
            pltpu.sync_copy(prow, sh.at[pl.ds(sid * L, L)])
            plsc.subcore_barrier()
            return phase2(sh, tpush, tpull, pcntf, plcnt, ac) + (it + 1,)

        def loop_body(carry):
            return half(half(carry, shpart2), shpart)

        fin = lax.while_loop(lambda cr: (cr[0] > 0) & (cr[12] < PP),
                             loop_body, carry0 + (jnp.int32(0),))
        tpush, tpull, pcntf, plcnt = fin[7], fin[8], fin[9], fin[10]

        push_v = (zero16 + tpush) / (zero16 + (pcntf + EPS))
        pull_v = (zero16 + tpull) / (zero16 + (plcnt.astype(jnp.float32) + EPS))

        @pl.when(sid == 0)
        def _():
            prow[...] = jnp.where(iota == 0, push_v,
                                  jnp.where(iota == 1, pull_v, 0.0))
            pltpu.sync_copy(prow, outh.at[cimg])

    return nms_kernel


@jax.jit
def kernel(pos_inds, pos_gt_index, gt_bboxes, bbox_preds, cls_scores, gt_labels):
    IMG, P, C = cls_scores.shape
    G = gt_bboxes.shape[1]
    SL = ((P + NTILES * L - 1) // (NTILES * L)) * L
    PP = SL * NTILES
    GIP = ((G * G + L - 1) // L) * L
    pad = PP - P

    bp = bbox_preds.astype(jnp.float32)
    x1 = jnp.pad(bp[..., 0], ((0, 0), (0, pad)))
    y1 = jnp.pad(bp[..., 1], ((0, 0), (0, pad)))
    x2 = jnp.pad(bp[..., 2], ((0, 0), (0, pad)))
    y2 = jnp.pad(bp[..., 3], ((0, 0), (0, pad)))
    clsp = jnp.pad(cls_scores.astype(jnp.float32), ((0, 0), (0, pad), (0, 0)))
    gti = jnp.pad(pos_gt_index.astype(jnp.int32), ((0, 0), (0, pad)))
    gtf = jnp.pad(gt_bboxes.astype(jnp.float32).reshape(IMG, G * 4),
                  ((0, 0), (0, 256 - G * 4)))
    gtl = jnp.pad(gt_labels.astype(jnp.int32), ((0, 0), (0, 64 - G)))

    out = _build(P, PP, G, C, GIP)(x1, y1, x2, y2, clsp, gti, gtf, gtl)

    push = (0.0 + out[0, 0] + out[1, 0]) / IMG * PUSH_W
    pull = (0.0 + out[0, 1] + out[1, 1]) / IMG * PULL_W
    return jnp.stack([push, pull])
